# baseline (speedup 1.0000x reference)
import functools

import jax
import jax.numpy as jnp
from jax import lax
from jax.experimental import pallas as pl
from jax.experimental.pallas import tpu as pltpu
from jax.experimental.pallas import tpu_sc as plsc

_VOCAB = 1000
_EMBED = 128
_B = 4096
_L = 20

_NC = 2
_NS = 16
_NW = _NC * _NS
_RPW = _B // _NW
_LANES = 16


_CHUNKS = [(1, 3), (4, 3), (7, 3), (10, 3), (13, 3), (16, 3), (19, 1)]


def _pool_body(ids_hbm, table_hbm, out_hbm, idsv, rows, acc, sem_a, sem0, sem1):
    wid = lax.axis_index("s") * _NC + lax.axis_index("c")
    sems = [sem0, sem1]
    pltpu.sync_copy(ids_hbm.at[wid], idsv)

    cp_acc = pltpu.async_copy(table_hbm.at[idsv.at[0]], acc, sem_a)

    def issue(ci):
        l0, n = _CHUNKS[ci]
        b = ci % 2
        return [
            pltpu.async_copy(table_hbm.at[idsv.at[l0 + j]], rows.at[b, j], sems[b])
            for j in range(n)
        ]

    cps = {0: issue(0), 1: issue(1)}
    cp_acc.wait()

    for ci, (l0, n) in enumerate(_CHUNKS):
        for cp in cps.pop(ci):
            cp.wait()
        b = ci % 2

        def body(r, carry, b=b, n=n):
            for c in range(_EMBED // _LANES):
                sl = pl.ds(c * _LANES, _LANES)
                v = rows[b, 0, r, sl]
                for j in range(1, n):
                    v = v + rows[b, j, r, sl]
                plsc.addupdate(acc.at[r, sl], v)
            return carry

        lax.fori_loop(0, _RPW, body, 0)
        if ci + 2 < len(_CHUNKS):
            cps[ci + 2] = issue(ci + 2)

    pltpu.sync_copy(acc, out_hbm.at[pl.ds(wid * _RPW, _RPW)])


@functools.cache
def _pool():
    return pl.kernel(
        _pool_body,
        mesh=plsc.VectorSubcoreMesh(core_axis_name="c", subcore_axis_name="s"),
        out_type=jax.ShapeDtypeStruct((_B, _EMBED), jnp.float32),
        scratch_types=[
            pltpu.VMEM((_L, _RPW), jnp.int32),
            pltpu.VMEM((2, 3, _RPW, _EMBED), jnp.float32),
            pltpu.VMEM((_RPW, _EMBED), jnp.float32),
            pltpu.SemaphoreType.DMA,
            pltpu.SemaphoreType.DMA,
            pltpu.SemaphoreType.DMA,
        ],
    )


_BB = 512


def _head_body(w_ref, p_ref, b_ref, out_ref):
    logits_t = lax.dot_general(
        w_ref[...],
        p_ref[...] * (1.0 / _L),
        (((1,), (1,)), ((), ())),
        preferred_element_type=jnp.float32,
    )
    out_ref[...] = logits_t + b_ref[...]


def _head(pooled, lm_w, lm_b_col):
    return pl.pallas_call(
        _head_body,
        grid=(_B // _BB,),
        in_specs=[
            pl.BlockSpec((_VOCAB, _EMBED), lambda i: (0, 0)),
            pl.BlockSpec((_BB, _EMBED), lambda i: (i, 0)),
            pl.BlockSpec((_VOCAB, 1), lambda i: (0, 0)),
        ],
        out_specs=pl.BlockSpec((_VOCAB, _BB), lambda i: (0, i)),
        out_shape=jax.ShapeDtypeStruct((_VOCAB, _B), jnp.float32),
    )(lm_w, pooled, lm_b_col)


_TROWS = _VOCAB // 8
_CPW2 = 4


def _rep_body(src_hbm, out_hbm, bufs, semr, semw0, semw1):
    wid = lax.axis_index("s") * _NC + lax.axis_index("c")
    sems = [semw0, semw1]

    for k in range(_CPW2):
        t = jnp.minimum(wid * _CPW2 + k, _TROWS - 1)
        b = k % 2
        if k >= 2:
            for _ in range(_L):
                pltpu.make_async_copy(
                    bufs.at[b], out_hbm.at[0, pl.ds(0, 8), :], sems[b]
                ).wait()
        pltpu.sync_copy(src_hbm.at[pl.ds(t * 8, 8)], bufs.at[b])
        for l in range(_L):
            pltpu.async_copy(bufs.at[b], out_hbm.at[l, pl.ds(t * 8, 8), :], sems[b])
    for b in range(2):
        for _ in range(_L):
            pltpu.make_async_copy(
                bufs.at[b], out_hbm.at[0, pl.ds(0, 8), :], sems[b]
            ).wait()


@functools.cache
def _rep():
    return pl.kernel(
        _rep_body,
        mesh=plsc.VectorSubcoreMesh(core_axis_name="c", subcore_axis_name="s"),
        out_type=jax.ShapeDtypeStruct((_L, _VOCAB, _B), jnp.float32),
        scratch_types=[
            pltpu.VMEM((2, 8, _B), jnp.float32),
            pltpu.SemaphoreType.DMA,
            pltpu.SemaphoreType.DMA,
            pltpu.SemaphoreType.DMA,
        ],
    )


def kernel(input_ids, emb_table, lm_w, lm_b):
    ids_blocks = (
        input_ids.astype(jnp.int32).T.reshape(_L, _NW, _RPW).transpose(1, 0, 2)
    )
    pooled = _pool()(ids_blocks, emb_table)
    logits_t = _head(pooled, lm_w, lm_b.reshape(_VOCAB, 1))
    out_lvb = _rep()(logits_t)
    return jnp.transpose(out_lvb, (2, 0, 1))


# speedup vs baseline: 1.0139x; 1.0139x over previous
import functools

import jax
import jax.numpy as jnp
from jax import lax
from jax.experimental import pallas as pl
from jax.experimental.pallas import tpu as pltpu
from jax.experimental.pallas import tpu_sc as plsc

_VOCAB = 1000
_EMBED = 128
_B = 4096
_L = 20

_NC = 2
_NS = 16
_NW = _NC * _NS
_RPW = _B // _NW
_LANES = 16


def _pool_body(ids_hbm, table_hbm, out_hbm, idsv, rows, acc, sem_a, sem0, sem1):
    wid = lax.axis_index("s") * _NC + lax.axis_index("c")
    pltpu.sync_copy(ids_hbm.at[wid], idsv)

    cp_acc = pltpu.async_copy(table_hbm.at[idsv.at[0]], acc, sem_a)
    prev = pltpu.async_copy(table_hbm.at[idsv.at[1]], rows.at[0], sem0)
    cp_acc.wait()

    for l in range(1, _L):
        cur = prev
        if l + 1 < _L:
            nb = l % 2
            prev = pltpu.async_copy(
                table_hbm.at[idsv.at[l + 1]], rows.at[nb], sem0 if nb == 0 else sem1
            )
        cur.wait()
        buf = (l - 1) % 2

        def body(r, carry, buf=buf):
            for c in range(_EMBED // _LANES):
                sl = pl.ds(c * _LANES, _LANES)
                plsc.addupdate(acc.at[r, sl], rows[buf, r, sl])
            return carry

        lax.fori_loop(0, _RPW, body, 0)

    pltpu.sync_copy(acc, out_hbm.at[pl.ds(wid * _RPW, _RPW)])


@functools.cache
def _pool():
    return pl.kernel(
        _pool_body,
        mesh=plsc.VectorSubcoreMesh(core_axis_name="c", subcore_axis_name="s"),
        out_type=jax.ShapeDtypeStruct((_B, _EMBED), jnp.float32),
        scratch_types=[
            pltpu.VMEM((_L, _RPW), jnp.int32),
            pltpu.VMEM((2, _RPW, _EMBED), jnp.float32),
            pltpu.VMEM((_RPW, _EMBED), jnp.float32),
            pltpu.SemaphoreType.DMA,
            pltpu.SemaphoreType.DMA,
            pltpu.SemaphoreType.DMA,
        ],
    )


_BB = 512


def _head_body(w_ref, p_ref, b_ref, out_ref):
    logits_t = lax.dot_general(
        w_ref[...],
        p_ref[...] * (1.0 / _L),
        (((1,), (1,)), ((), ())),
        preferred_element_type=jnp.float32,
    )
    out_ref[...] = logits_t + b_ref[...]


def _head(pooled, lm_w, lm_b_col):
    return pl.pallas_call(
        _head_body,
        grid=(_B // _BB,),
        in_specs=[
            pl.BlockSpec((_VOCAB, _EMBED), lambda i: (0, 0)),
            pl.BlockSpec((_BB, _EMBED), lambda i: (i, 0)),
            pl.BlockSpec((_VOCAB, 1), lambda i: (0, 0)),
        ],
        out_specs=pl.BlockSpec((_VOCAB, _BB), lambda i: (0, i)),
        out_shape=jax.ShapeDtypeStruct((_VOCAB, _B), jnp.float32),
    )(lm_w, pooled, lm_b_col)


_TROWS = _VOCAB // 8
_CPW2 = 4


def _rep_body(src_hbm, out_hbm, bufs, semr, semw0, semw1):
    wid = lax.axis_index("s") * _NC + lax.axis_index("c")
    sems = [semw0, semw1]

    for k in range(_CPW2):
        t = jnp.minimum(wid * _CPW2 + k, _TROWS - 1)
        b = k % 2
        if k >= 2:
            for _ in range(_L):
                pltpu.make_async_copy(
                    bufs.at[b], out_hbm.at[0, pl.ds(0, 8), :], sems[b]
                ).wait()
        pltpu.sync_copy(src_hbm.at[pl.ds(t * 8, 8)], bufs.at[b])
        for l in range(_L):
            pltpu.async_copy(bufs.at[b], out_hbm.at[l, pl.ds(t * 8, 8), :], sems[b])
    for b in range(2):
        for _ in range(_L):
            pltpu.make_async_copy(
                bufs.at[b], out_hbm.at[0, pl.ds(0, 8), :], sems[b]
            ).wait()


@functools.cache
def _rep():
    return pl.kernel(
        _rep_body,
        mesh=plsc.VectorSubcoreMesh(core_axis_name="c", subcore_axis_name="s"),
        out_type=jax.ShapeDtypeStruct((_L, _VOCAB, _B), jnp.float32),
        scratch_types=[
            pltpu.VMEM((2, 8, _B), jnp.float32),
            pltpu.SemaphoreType.DMA,
            pltpu.SemaphoreType.DMA,
            pltpu.SemaphoreType.DMA,
        ],
    )


def kernel(input_ids, emb_table, lm_w, lm_b):
    ids_blocks = (
        input_ids.astype(jnp.int32).T.reshape(_L, _NW, _RPW).transpose(1, 0, 2)
    )
    pooled = _pool()(ids_blocks, emb_table)
    logits_t = _head(pooled, lm_w, lm_b.reshape(_VOCAB, 1))
    out_lvb = _rep()(logits_t)
    return jnp.transpose(out_lvb, (2, 0, 1))


# speedup vs baseline: 1.0146x; 1.0007x over previous
import functools

import jax
import jax.numpy as jnp
from jax import lax
from jax.experimental import pallas as pl
from jax.experimental.pallas import tpu as pltpu
from jax.experimental.pallas import tpu_sc as plsc

_VOCAB = 1000
_EMBED = 128
_B = 4096
_L = 20

_NC = 2
_NS = 16
_NW = _NC * _NS
_RPW = _B // _NW
_LANES = 16


def _pool_body(ids_hbm, table_hbm, out_hbm, idsv, rows, acc, sem_a, sem0, sem1):
    wid = lax.axis_index("s") * _NC + lax.axis_index("c")
    pltpu.sync_copy(ids_hbm.at[wid], idsv)

    cp_acc = pltpu.async_copy(table_hbm.at[idsv.at[0]], acc, sem_a)
    prev = pltpu.async_copy(table_hbm.at[idsv.at[1]], rows.at[0], sem0)
    cp_acc.wait()

    for l in range(1, _L):
        cur = prev
        if l + 1 < _L:
            nb = l % 2
            prev = pltpu.async_copy(
                table_hbm.at[idsv.at[l + 1]], rows.at[nb], sem0 if nb == 0 else sem1
            )
        cur.wait()
        buf = (l - 1) % 2

        def body(r, carry, buf=buf):
            for c in range(_EMBED // _LANES):
                sl = pl.ds(c * _LANES, _LANES)
                plsc.addupdate(acc.at[r, sl], rows[buf, r, sl])
            return carry

        lax.fori_loop(0, _RPW, body, 0)

    pltpu.sync_copy(acc, out_hbm.at[pl.ds(wid * _RPW, _RPW)])


@functools.cache
def _pool():
    return pl.kernel(
        _pool_body,
        mesh=plsc.VectorSubcoreMesh(core_axis_name="c", subcore_axis_name="s"),
        out_type=jax.ShapeDtypeStruct((_B, _EMBED), jnp.float32),
        scratch_types=[
            pltpu.VMEM((_L, _RPW), jnp.int32),
            pltpu.VMEM((2, _RPW, _EMBED), jnp.float32),
            pltpu.VMEM((_RPW, _EMBED), jnp.float32),
            pltpu.SemaphoreType.DMA,
            pltpu.SemaphoreType.DMA,
            pltpu.SemaphoreType.DMA,
        ],
    )


_BB = 512


def _head_body(w_ref, p_ref, b_ref, out_ref):
    logits_t = lax.dot_general(
        w_ref[...],
        p_ref[...] * (1.0 / _L),
        (((1,), (1,)), ((), ())),
        preferred_element_type=jnp.float32,
    )
    out_ref[...] = logits_t + b_ref[...]


def _head(pooled, lm_w, lm_b_col):
    return pl.pallas_call(
        _head_body,
        grid=(_B // _BB,),
        in_specs=[
            pl.BlockSpec((_VOCAB, _EMBED), lambda i: (0, 0)),
            pl.BlockSpec((_BB, _EMBED), lambda i: (i, 0)),
            pl.BlockSpec((_VOCAB, 1), lambda i: (0, 0)),
        ],
        out_specs=pl.BlockSpec((_VOCAB, _BB), lambda i: (0, i)),
        out_shape=jax.ShapeDtypeStruct((_VOCAB, _B), jnp.float32),
    )(lm_w, pooled, lm_b_col)


_TROWS = _VOCAB // 8
_CPW2 = 4


def _rep_body(src_hbm, out_hbm, bufs, semr, semw0, semw1):
    wid = lax.axis_index("s") * _NC + lax.axis_index("c")
    sems = [semw0, semw1]

    for k in range(_CPW2):
        t = wid + _NW * k
        b = k % 2
        if k >= 2:
            for _ in range(_L):
                pltpu.make_async_copy(
                    bufs.at[b], out_hbm.at[0, pl.ds(0, 8), :], sems[b]
                ).wait()

        def work(t=t, b=b):
            pltpu.sync_copy(src_hbm.at[pl.ds(t * 8, 8)], bufs.at[b])
            for l in range(_L):
                pltpu.async_copy(
                    bufs.at[b], out_hbm.at[l, pl.ds(t * 8, 8), :], sems[b]
                )

        if k < _CPW2 - 1:
            work()
        else:
            pl.when(t < _TROWS)(work)

    for _ in range(_L):
        pltpu.make_async_copy(
            bufs.at[0], out_hbm.at[0, pl.ds(0, 8), :], sems[0]
        ).wait()

    @pl.when(wid + _NW * (_CPW2 - 1) < _TROWS)
    def _():
        for _ in range(_L):
            pltpu.make_async_copy(
                bufs.at[1], out_hbm.at[0, pl.ds(0, 8), :], sems[1]
            ).wait()


@functools.cache
def _rep():
    return pl.kernel(
        _rep_body,
        mesh=plsc.VectorSubcoreMesh(core_axis_name="c", subcore_axis_name="s"),
        out_type=jax.ShapeDtypeStruct((_L, _VOCAB, _B), jnp.float32),
        scratch_types=[
            pltpu.VMEM((2, 8, _B), jnp.float32),
            pltpu.SemaphoreType.DMA,
            pltpu.SemaphoreType.DMA,
            pltpu.SemaphoreType.DMA,
        ],
    )


def kernel(input_ids, emb_table, lm_w, lm_b):
    ids_blocks = (
        input_ids.astype(jnp.int32).T.reshape(_L, _NW, _RPW).transpose(1, 0, 2)
    )
    pooled = _pool()(ids_blocks, emb_table)
    logits_t = _head(pooled, lm_w, lm_b.reshape(_VOCAB, 1))
    out_lvb = _rep()(logits_t)
    return jnp.transpose(out_lvb, (2, 0, 1))
